# Initial kernel scaffold; baseline (speedup 1.0000x reference)
import functools

import jax
import jax.numpy as jnp
from jax.experimental import pallas as pl
from jax.experimental.pallas import tpu as pltpu

HID = 64
EMB = 8
CT = 26
LAYERS = 4
ROW_BLK = 2500


def _linear_body(x_ref, w_ref, b_ref, o_ref, *, act):
    y = jnp.dot(x_ref[...], w_ref[...], preferred_element_type=jnp.float32) + b_ref[...]
    if act:
        y = jnp.maximum(y, 0.0)
    o_ref[...] = y


def _linear(x, w, b, act):
    n, k = x.shape
    o = w.shape[1]
    grid = n // ROW_BLK
    return pl.pallas_call(
        functools.partial(_linear_body, act=act),
        grid=(grid,),
        in_specs=[
            pl.BlockSpec((ROW_BLK, k), lambda i: (i, 0)),
            pl.BlockSpec((k, o), lambda i: (0, 0)),
            pl.BlockSpec((1, o), lambda i: (0, 0)),
        ],
        out_specs=pl.BlockSpec((ROW_BLK, o), lambda i: (i, 0)),
        out_shape=jax.ShapeDtypeStruct((n, o), jnp.float32),
    )(x, w, b.reshape(1, o))


def _update_body(h_ref, c_ref, w_ref, b_ref, g_ref, beta_ref, o_ref):
    x = h_ref[...] + jnp.dot(c_ref[...], w_ref[...], preferred_element_type=jnp.float32) + b_ref[...]
    m = jnp.mean(x, axis=-1, keepdims=True)
    v = jnp.mean((x - m) ** 2, axis=-1, keepdims=True)
    y = (x - m) * jax.lax.rsqrt(v + 1e-5) * g_ref[...] + beta_ref[...]
    o_ref[...] = jnp.maximum(y, 0.0)


def _update(h, comb, w, b, g, beta):
    n = h.shape[0]
    grid = n // ROW_BLK
    return pl.pallas_call(
        _update_body,
        grid=(grid,),
        in_specs=[
            pl.BlockSpec((ROW_BLK, HID), lambda i: (i, 0)),
            pl.BlockSpec((ROW_BLK, 3 * HID), lambda i: (i, 0)),
            pl.BlockSpec((3 * HID, HID), lambda i: (0, 0)),
            pl.BlockSpec((1, HID), lambda i: (0, 0)),
            pl.BlockSpec((1, HID), lambda i: (0, 0)),
            pl.BlockSpec((1, HID), lambda i: (0, 0)),
        ],
        out_specs=pl.BlockSpec((ROW_BLK, HID), lambda i: (i, 0)),
        out_shape=jax.ShapeDtypeStruct((n, HID), jnp.float32),
    )(h, comb, w, b.reshape(1, HID), g.reshape(1, HID), beta.reshape(1, HID))


def _cell_mlp_body(h_ref, e_ref, w1a_ref, w1b_ref, b1_ref, w2_ref, b2_ref, o_ref):
    t = (jnp.dot(h_ref[...], w1a_ref[...], preferred_element_type=jnp.float32)
         + jnp.dot(e_ref[...], w1b_ref[...], preferred_element_type=jnp.float32)
         + b1_ref[...])
    t = jnp.maximum(t, 0.0)
    o_ref[...] = jnp.dot(t, w2_ref[...], preferred_element_type=jnp.float32) + b2_ref[...]


def _cell_mlp(cell_h, cte, w1, b1, w2, b2):
    n = cell_h.shape[0]
    grid = n // ROW_BLK
    w1a = w1[:HID]
    w1b = w1[HID:]
    return pl.pallas_call(
        _cell_mlp_body,
        grid=(grid,),
        in_specs=[
            pl.BlockSpec((ROW_BLK, HID), lambda i: (i, 0)),
            pl.BlockSpec((ROW_BLK, EMB), lambda i: (i, 0)),
            pl.BlockSpec((HID, HID), lambda i: (0, 0)),
            pl.BlockSpec((EMB, HID), lambda i: (0, 0)),
            pl.BlockSpec((1, HID), lambda i: (0, 0)),
            pl.BlockSpec((HID, HID), lambda i: (0, 0)),
            pl.BlockSpec((1, HID), lambda i: (0, 0)),
        ],
        out_specs=pl.BlockSpec((ROW_BLK, HID), lambda i: (i, 0)),
        out_shape=jax.ShapeDtypeStruct((n, HID), jnp.float32),
    )(cell_h, cte, w1a, w1b, b1.reshape(1, HID), w2, b2.reshape(1, HID))


def _aggregate(src_x, edge_index, out_size):
    src = edge_index[0]
    dst = edge_index[1]
    msg = src_x[src]
    deg = jnp.maximum(jnp.bincount(src, length=src_x.shape[0]), 1)
    norm = 1.0 / deg[src].astype(jnp.float32)
    agg_mean = jax.ops.segment_sum(msg * norm[:, None], dst, num_segments=out_size)
    cnt = jnp.bincount(dst, length=out_size)
    agg_max = jax.ops.segment_max(msg, dst, num_segments=out_size)
    agg_max = jnp.where(cnt[:, None] > 0, agg_max, 0.0)
    agg_min = jax.ops.segment_min(msg, dst, num_segments=out_size)
    agg_min = jnp.where(cnt[:, None] > 0, agg_min, 0.0)
    return jnp.concatenate([agg_mean, agg_max, agg_min], axis=1)


def kernel(net_x, pin_in_x, pin_out_x, cell_x, edge_net_pin_in, edge_pin_in_cell,
           edge_cell_pin_out, edge_pin_out_net, params):
    p = params
    net_h = _linear(net_x, p['net_lin_w'], p['net_lin_b'], act=True)
    pin_in_h = _linear(pin_in_x, p['pin_in_lin_w'], p['pin_in_lin_b'], act=True)
    pin_out_h = _linear(pin_out_x, p['pin_out_lin_w'], p['pin_out_lin_b'], act=True)
    cell_h = _linear(cell_x[:, CT:], p['cell_struct_w'], p['cell_struct_b'], act=True)
    cte = _linear(cell_x[:, :CT], p['cte_w'], p['cte_b'], act=False)

    n_pin = pin_in_h.shape[0]
    n_cell = cell_h.shape[0]
    n_net = net_h.shape[0]

    for _ in range(LAYERS):
        comb = _aggregate(net_h, edge_net_pin_in, n_pin)
        pin_in_h = _update(pin_in_h, comb, p['pin_in_up_w'], p['pin_in_up_b'],
                           p['pin_in_norm_g'], p['pin_in_norm_b'])
        comb = _aggregate(pin_in_h, edge_pin_in_cell, n_cell)
        cell_h = _update(cell_h, comb, p['cell_up_w'], p['cell_up_b'],
                         p['cell_norm_g'], p['cell_norm_b'])
        cell_out = _cell_mlp(cell_h, cte, p['mlp1_w'], p['mlp1_b'], p['mlp2_w'], p['mlp2_b'])
        comb = _aggregate(cell_out, edge_cell_pin_out, n_pin)
        pin_out_h = _update(pin_out_h, comb, p['pin_out_up_w'], p['pin_out_up_b'],
                            p['pin_out_norm_g'], p['pin_out_norm_b'])
        comb = _aggregate(pin_out_h, edge_pin_out_net, n_net)
        net_h = _update(net_h, comb, p['net_up_w'], p['net_up_b'],
                        p['net_norm_g'], p['net_norm_b'])

    return _linear(net_h, p['out_w'], p['out_b'], act=False)


# speedup vs baseline: 1.0189x; 1.0189x over previous
import functools

import jax
import jax.numpy as jnp
from jax.experimental import pallas as pl
from jax.experimental.pallas import tpu as pltpu

HID = 64
EMB = 8
CT = 26
LAYERS = 4
ROW_BLK = 2000


def _linear_body(x_ref, w_ref, b_ref, o_ref, *, act):
    y = jnp.dot(x_ref[...], w_ref[...], preferred_element_type=jnp.float32) + b_ref[...]
    if act:
        y = jnp.maximum(y, 0.0)
    o_ref[...] = y


def _linear(x, w, b, act):
    n, k = x.shape
    o = w.shape[1]
    grid = n // ROW_BLK
    return pl.pallas_call(
        functools.partial(_linear_body, act=act),
        grid=(grid,),
        in_specs=[
            pl.BlockSpec((ROW_BLK, k), lambda i: (i, 0)),
            pl.BlockSpec((k, o), lambda i: (0, 0)),
            pl.BlockSpec((1, o), lambda i: (0, 0)),
        ],
        out_specs=pl.BlockSpec((ROW_BLK, o), lambda i: (i, 0)),
        out_shape=jax.ShapeDtypeStruct((n, o), jnp.float32),
    )(x, w, b.reshape(1, o))


def _update_body(h_ref, c_ref, w_ref, b_ref, g_ref, beta_ref, o_ref):
    x = h_ref[...] + jnp.dot(c_ref[...], w_ref[...], preferred_element_type=jnp.float32) + b_ref[...]
    m = jnp.mean(x, axis=-1, keepdims=True)
    v = jnp.mean((x - m) ** 2, axis=-1, keepdims=True)
    y = (x - m) * jax.lax.rsqrt(v + 1e-5) * g_ref[...] + beta_ref[...]
    o_ref[...] = jnp.maximum(y, 0.0)


def _update(h, comb, w, b, g, beta):
    n = h.shape[0]
    grid = n // ROW_BLK
    return pl.pallas_call(
        _update_body,
        grid=(grid,),
        in_specs=[
            pl.BlockSpec((ROW_BLK, HID), lambda i: (i, 0)),
            pl.BlockSpec((ROW_BLK, 3 * HID), lambda i: (i, 0)),
            pl.BlockSpec((3 * HID, HID), lambda i: (0, 0)),
            pl.BlockSpec((1, HID), lambda i: (0, 0)),
            pl.BlockSpec((1, HID), lambda i: (0, 0)),
            pl.BlockSpec((1, HID), lambda i: (0, 0)),
        ],
        out_specs=pl.BlockSpec((ROW_BLK, HID), lambda i: (i, 0)),
        out_shape=jax.ShapeDtypeStruct((n, HID), jnp.float32),
    )(h, comb, w, b.reshape(1, HID), g.reshape(1, HID), beta.reshape(1, HID))


def _cell_mlp_body(h_ref, e_ref, w1a_ref, w1b_ref, b1_ref, w2_ref, b2_ref, o_ref):
    t = (jnp.dot(h_ref[...], w1a_ref[...], preferred_element_type=jnp.float32)
         + jnp.dot(e_ref[...], w1b_ref[...], preferred_element_type=jnp.float32)
         + b1_ref[...])
    t = jnp.maximum(t, 0.0)
    o_ref[...] = jnp.dot(t, w2_ref[...], preferred_element_type=jnp.float32) + b2_ref[...]


def _cell_mlp(cell_h, cte, w1, b1, w2, b2):
    n = cell_h.shape[0]
    grid = n // ROW_BLK
    w1a = w1[:HID]
    w1b = w1[HID:]
    return pl.pallas_call(
        _cell_mlp_body,
        grid=(grid,),
        in_specs=[
            pl.BlockSpec((ROW_BLK, HID), lambda i: (i, 0)),
            pl.BlockSpec((ROW_BLK, EMB), lambda i: (i, 0)),
            pl.BlockSpec((HID, HID), lambda i: (0, 0)),
            pl.BlockSpec((EMB, HID), lambda i: (0, 0)),
            pl.BlockSpec((1, HID), lambda i: (0, 0)),
            pl.BlockSpec((HID, HID), lambda i: (0, 0)),
            pl.BlockSpec((1, HID), lambda i: (0, 0)),
        ],
        out_specs=pl.BlockSpec((ROW_BLK, HID), lambda i: (i, 0)),
        out_shape=jax.ShapeDtypeStruct((n, HID), jnp.float32),
    )(cell_h, cte, w1a, w1b, b1.reshape(1, HID), w2, b2.reshape(1, HID))


def _aggregate(src_x, edge_index, out_size):
    src = edge_index[0]
    dst = edge_index[1]
    msg = src_x[src]
    deg = jnp.maximum(jnp.bincount(src, length=src_x.shape[0]), 1)
    norm = 1.0 / deg[src].astype(jnp.float32)
    agg_mean = jax.ops.segment_sum(msg * norm[:, None], dst, num_segments=out_size)
    cnt = jnp.bincount(dst, length=out_size)
    agg_max = jax.ops.segment_max(msg, dst, num_segments=out_size)
    agg_max = jnp.where(cnt[:, None] > 0, agg_max, 0.0)
    agg_min = jax.ops.segment_min(msg, dst, num_segments=out_size)
    agg_min = jnp.where(cnt[:, None] > 0, agg_min, 0.0)
    return jnp.concatenate([agg_mean, agg_max, agg_min], axis=1)


def kernel(net_x, pin_in_x, pin_out_x, cell_x, edge_net_pin_in, edge_pin_in_cell,
           edge_cell_pin_out, edge_pin_out_net, params):
    p = params
    net_h = _linear(net_x, p['net_lin_w'], p['net_lin_b'], act=True)
    pin_in_h = _linear(pin_in_x, p['pin_in_lin_w'], p['pin_in_lin_b'], act=True)
    pin_out_h = _linear(pin_out_x, p['pin_out_lin_w'], p['pin_out_lin_b'], act=True)
    cell_h = _linear(cell_x[:, CT:], p['cell_struct_w'], p['cell_struct_b'], act=True)
    cte = _linear(cell_x[:, :CT], p['cte_w'], p['cte_b'], act=False)

    n_pin = pin_in_h.shape[0]
    n_cell = cell_h.shape[0]
    n_net = net_h.shape[0]

    for _ in range(LAYERS):
        comb = _aggregate(net_h, edge_net_pin_in, n_pin)
        pin_in_h = _update(pin_in_h, comb, p['pin_in_up_w'], p['pin_in_up_b'],
                           p['pin_in_norm_g'], p['pin_in_norm_b'])
        comb = _aggregate(pin_in_h, edge_pin_in_cell, n_cell)
        cell_h = _update(cell_h, comb, p['cell_up_w'], p['cell_up_b'],
                         p['cell_norm_g'], p['cell_norm_b'])
        cell_out = _cell_mlp(cell_h, cte, p['mlp1_w'], p['mlp1_b'], p['mlp2_w'], p['mlp2_b'])
        comb = _aggregate(cell_out, edge_cell_pin_out, n_pin)
        pin_out_h = _update(pin_out_h, comb, p['pin_out_up_w'], p['pin_out_up_b'],
                            p['pin_out_norm_g'], p['pin_out_norm_b'])
        comb = _aggregate(pin_out_h, edge_pin_out_net, n_net)
        net_h = _update(net_h, comb, p['net_up_w'], p['net_up_b'],
                        p['net_norm_g'], p['net_norm_b'])

    return _linear(net_h, p['out_w'], p['out_b'], act=False)


# speedup vs baseline: 2.6836x; 2.6338x over previous
import functools

import jax
import jax.numpy as jnp
from jax import lax
from jax.experimental import pallas as pl
from jax.experimental.pallas import tpu as pltpu
from jax.experimental.pallas import tpu_sc as plsc

HID = 64
EMB = 8
CT = 26
LAYERS = 4
ROW_BLK = 2000

NW = 32


WIDE = 128


def _linear_body(x_ref, w_ref, b_ref, o_ref, *, act, k, o):
    x = x_ref[...][:, :k]
    y = jnp.dot(x, w_ref[...], preferred_element_type=jnp.float32) + b_ref[...]
    if act:
        y = jnp.maximum(y, 0.0)
    o_ref[:, :o] = y


def _linear(x, w, b, act, wide_out=False):
    n, kin = x.shape
    k, o = w.shape
    o_out = WIDE if wide_out else o
    grid = n // ROW_BLK
    return pl.pallas_call(
        functools.partial(_linear_body, act=act, k=k, o=o),
        grid=(grid,),
        in_specs=[
            pl.BlockSpec((ROW_BLK, kin), lambda i: (i, 0)),
            pl.BlockSpec((k, o), lambda i: (0, 0)),
            pl.BlockSpec((1, o), lambda i: (0, 0)),
        ],
        out_specs=pl.BlockSpec((ROW_BLK, o_out), lambda i: (i, 0)),
        out_shape=jax.ShapeDtypeStruct((n, o_out), jnp.float32),
    )(x, w, b.reshape(1, o))


def _update_body(h_ref, c_ref, w_ref, b_ref, g_ref, beta_ref, o_ref):
    h = h_ref[...][:, :HID]
    x = h + jnp.dot(c_ref[...], w_ref[...], preferred_element_type=jnp.float32) + b_ref[...]
    m = jnp.mean(x, axis=-1, keepdims=True)
    v = jnp.mean((x - m) ** 2, axis=-1, keepdims=True)
    y = (x - m) * jax.lax.rsqrt(v + 1e-5) * g_ref[...] + beta_ref[...]
    o_ref[:, :HID] = jnp.maximum(y, 0.0)


def _update(h, comb, w, b, g, beta):
    n = h.shape[0]
    grid = n // ROW_BLK
    return pl.pallas_call(
        _update_body,
        grid=(grid,),
        in_specs=[
            pl.BlockSpec((ROW_BLK, WIDE), lambda i: (i, 0)),
            pl.BlockSpec((ROW_BLK, 3 * HID), lambda i: (i, 0)),
            pl.BlockSpec((3 * HID, HID), lambda i: (0, 0)),
            pl.BlockSpec((1, HID), lambda i: (0, 0)),
            pl.BlockSpec((1, HID), lambda i: (0, 0)),
            pl.BlockSpec((1, HID), lambda i: (0, 0)),
        ],
        out_specs=pl.BlockSpec((ROW_BLK, WIDE), lambda i: (i, 0)),
        out_shape=jax.ShapeDtypeStruct((n, WIDE), jnp.float32),
    )(h, comb, w, b.reshape(1, HID), g.reshape(1, HID), beta.reshape(1, HID))


def _cell_mlp_body(h_ref, e_ref, w1a_ref, w1b_ref, b1_ref, w2_ref, b2_ref, o_ref):
    h = h_ref[...][:, :HID]
    t = (jnp.dot(h, w1a_ref[...], preferred_element_type=jnp.float32)
         + jnp.dot(e_ref[...], w1b_ref[...], preferred_element_type=jnp.float32)
         + b1_ref[...])
    t = jnp.maximum(t, 0.0)
    o_ref[:, :HID] = jnp.dot(t, w2_ref[...], preferred_element_type=jnp.float32) + b2_ref[...]


def _cell_mlp(cell_h, cte, w1, b1, w2, b2):
    n = cell_h.shape[0]
    grid = n // ROW_BLK
    w1a = w1[:HID]
    w1b = w1[HID:]
    return pl.pallas_call(
        _cell_mlp_body,
        grid=(grid,),
        in_specs=[
            pl.BlockSpec((ROW_BLK, WIDE), lambda i: (i, 0)),
            pl.BlockSpec((ROW_BLK, EMB), lambda i: (i, 0)),
            pl.BlockSpec((HID, HID), lambda i: (0, 0)),
            pl.BlockSpec((EMB, HID), lambda i: (0, 0)),
            pl.BlockSpec((1, HID), lambda i: (0, 0)),
            pl.BlockSpec((HID, HID), lambda i: (0, 0)),
            pl.BlockSpec((1, HID), lambda i: (0, 0)),
        ],
        out_specs=pl.BlockSpec((ROW_BLK, WIDE), lambda i: (i, 0)),
        out_shape=jax.ShapeDtypeStruct((n, WIDE), jnp.float32),
    )(cell_h, cte, w1a, w1b, b1.reshape(1, HID), w2, b2.reshape(1, HID))


def _make_sc_agg(n_src, d_tile, c_edge, r_out, num_cores=None, num_subcores=None,
                 interpret=False):
    mesh_kw = {}
    if num_cores is not None:
        mesh_kw = dict(num_cores=num_cores, num_subcores=num_subcores)
    mesh = plsc.VectorSubcoreMesh(core_axis_name="c", subcore_axis_name="s", **mesh_kw)
    n_pad = mesh.num_cores * mesh.num_subcores * d_tile
    rs_len = d_tile + 16

    @functools.partial(
        pl.kernel,
        out_type=jax.ShapeDtypeStruct((n_pad, 192), jnp.float32),
        mesh=mesh,
        interpret=interpret,
        compiler_params=None if interpret else pltpu.CompilerParams(
            needs_layout_passes=False, use_tc_tiling_on_sc=False),
        scratch_types=[
            pltpu.VMEM((2, 2, c_edge), jnp.int32),
            pltpu.VMEM((2, c_edge, WIDE), jnp.float32),
            pltpu.VMEM((rs_len,), jnp.int32),
            pltpu.VMEM((r_out, 192), jnp.float32),
            pltpu.SemaphoreType.DMA,
            pltpu.SemaphoreType.DMA,
        ],
    )
    def agg(ed, rs, table, out, env_v, gat_v, rs_v, out_v, sem0, sem1):
        if interpret:
            wid = jnp.int32(0)
        else:
            wid = lax.axis_index("s") * mesh.num_cores + lax.axis_index("c")
        r0 = wid * d_tile
        pltpu.sync_copy(rs.at[pl.ds(r0, rs_len)], rs_v)
        e0 = rs_v[pl.ds(0, 16)][0]
        e1 = rs_v[pl.ds(d_tile, 16)][0]
        eb = (e0 // 128) * 128
        n_chunks = jnp.where(e1 > e0, (e1 - eb + c_edge - 1) // c_edge, 0)

        def idx_of(parity):
            return env_v[parity, 0] if interpret else env_v.at[parity, 0]

        def start(g, parity):
            env = env_v.at[parity]
            pltpu.sync_copy(ed.at[:, pl.ds(eb + g * c_edge, c_edge)], env)
            sem = sem0 if parity == 0 else sem1
            pltpu.async_copy(table.at[idx_of(parity)], gat_v.at[parity], sem)

        def start_dyn(g):
            @pl.when(lax.rem(g, 2) == 0)
            def _():
                start(g, 0)

            @pl.when(lax.rem(g, 2) == 1)
            def _():
                start(g, 1)

        def wait_slot(parity):
            sem = sem0 if parity == 0 else sem1
            pltpu.make_async_copy(table.at[idx_of(parity)],
                                  gat_v.at[parity], sem).wait()

        @pl.when(n_chunks > 0)
        def _():
            start(0, 0)

        zeros = jnp.zeros((16,), jnp.float32)
        ninf = jnp.full((16,), -jnp.inf, jnp.float32)
        pinf = jnp.full((16,), jnp.inf, jnp.float32)

        def row_body(i, cur0):
            rv = rs_v[pl.ds(i, 16)]
            s_e = rv[0]
            t_e = rv[1]

            def edge_body(e, carry):
                (cur, a0, a1, a2, a3, x0, x1, x2, x3, n0, n1, n2, n3) = carry
                adv = e >= eb + (cur + 1) * c_edge

                @pl.when(adv)
                def _():
                    g_new = cur + 1

                    @pl.when(g_new + 1 < n_chunks)
                    def _():
                        start_dyn(g_new + 1)

                    @pl.when(lax.rem(g_new, 2) == 0)
                    def _():
                        wait_slot(0)

                    @pl.when(lax.rem(g_new, 2) == 1)
                    def _():
                        wait_slot(1)

                cur = jnp.where(adv, cur + 1, cur)
                el = e - eb - cur * c_edge
                slot = lax.rem(cur, 2)
                if interpret:
                    s = lax.bitcast_convert_type(env_v[slot, 1, el], jnp.float32)
                else:
                    splat = lambda v: jnp.full((16,), v, jnp.int32)
                    s_bits = plsc.load_gather(env_v, [splat(slot), splat(1), splat(el)])
                    s = plsc.bitcast(s_bits, jnp.float32)
                m0 = gat_v[slot, el, pl.ds(0, 16)]
                m1 = gat_v[slot, el, pl.ds(16, 16)]
                m2 = gat_v[slot, el, pl.ds(32, 16)]
                m3 = gat_v[slot, el, pl.ds(48, 16)]
                return (cur,
                        a0 + m0 * s, a1 + m1 * s, a2 + m2 * s, a3 + m3 * s,
                        jnp.maximum(x0, m0), jnp.maximum(x1, m1),
                        jnp.maximum(x2, m2), jnp.maximum(x3, m3),
                        jnp.minimum(n0, m0), jnp.minimum(n1, m1),
                        jnp.minimum(n2, m2), jnp.minimum(n3, m3))

            init = (cur0, zeros, zeros, zeros, zeros,
                    ninf, ninf, ninf, ninf, pinf, pinf, pinf, pinf)
            res = lax.fori_loop(s_e, t_e, edge_body, init)
            cur = res[0]
            has = t_e > s_e
            il = lax.rem(i, r_out)
            for j in range(4):
                out_v[il, pl.ds(16 * j, 16)] = res[1 + j]
                out_v[il, pl.ds(64 + 16 * j, 16)] = jnp.where(has, res[5 + j], 0.0)
                out_v[il, pl.ds(128 + 16 * j, 16)] = jnp.where(has, res[9 + j], 0.0)

            @pl.when(il == r_out - 1)
            def _():
                row = pl.multiple_of(r0 + i - (r_out - 1), r_out)
                pltpu.sync_copy(out_v, out.at[pl.ds(row, r_out)])

            return cur

        lax.fori_loop(0, d_tile, row_body, jnp.int32(-1))

    return agg


D_TILE = 1664
C_EDGE = 256
R_OUT = 128
N_PAD = NW * D_TILE


def _prep_relation(edge_index, n_src):
    src = edge_index[0]
    dst = edge_index[1]
    deg = jnp.maximum(jnp.bincount(src, length=n_src), 1)
    inv = (1.0 / deg.astype(jnp.float32))[src]
    order = jnp.argsort(dst)
    src_s = src[order]
    nrm_bits = lax.bitcast_convert_type(inv[order], jnp.int32)
    dst_s = dst[order]
    ed = jnp.stack([jnp.pad(src_s, (0, C_EDGE)), jnp.pad(nrm_bits, (0, C_EDGE))])
    rs = jnp.searchsorted(dst_s, jnp.arange(N_PAD + 64, dtype=jnp.int32),
                          side='left').astype(jnp.int32)
    return ed, rs


_sc_agg = None


def _aggregate(src_x, prep):
    global _sc_agg
    if _sc_agg is None:
        _sc_agg = _make_sc_agg(src_x.shape[0], D_TILE, C_EDGE, R_OUT)
    ed, rs = prep
    return _sc_agg(ed, rs, src_x)


def kernel(net_x, pin_in_x, pin_out_x, cell_x, edge_net_pin_in, edge_pin_in_cell,
           edge_cell_pin_out, edge_pin_out_net, params):
    p = params
    net_h = _linear(net_x, p['net_lin_w'], p['net_lin_b'], act=True, wide_out=True)
    pin_in_h = _linear(pin_in_x, p['pin_in_lin_w'], p['pin_in_lin_b'], act=True, wide_out=True)
    pin_out_h = _linear(pin_out_x, p['pin_out_lin_w'], p['pin_out_lin_b'], act=True, wide_out=True)
    cell_h = _linear(cell_x[:, CT:], p['cell_struct_w'], p['cell_struct_b'], act=True, wide_out=True)
    cte = _linear(cell_x[:, :CT], p['cte_w'], p['cte_b'], act=False)

    prep_npi = _prep_relation(edge_net_pin_in, net_h.shape[0])
    prep_pic = _prep_relation(edge_pin_in_cell, pin_in_h.shape[0])
    prep_cpo = _prep_relation(edge_cell_pin_out, cell_h.shape[0])
    prep_pon = _prep_relation(edge_pin_out_net, pin_out_h.shape[0])

    for _ in range(LAYERS):
        comb = _aggregate(net_h, prep_npi)
        pin_in_h = _update(pin_in_h, comb, p['pin_in_up_w'], p['pin_in_up_b'],
                           p['pin_in_norm_g'], p['pin_in_norm_b'])
        comb = _aggregate(pin_in_h, prep_pic)
        cell_h = _update(cell_h, comb, p['cell_up_w'], p['cell_up_b'],
                         p['cell_norm_g'], p['cell_norm_b'])
        cell_out = _cell_mlp(cell_h, cte, p['mlp1_w'], p['mlp1_b'], p['mlp2_w'], p['mlp2_b'])
        comb = _aggregate(cell_out, prep_cpo)
        pin_out_h = _update(pin_out_h, comb, p['pin_out_up_w'], p['pin_out_up_b'],
                            p['pin_out_norm_g'], p['pin_out_norm_b'])
        comb = _aggregate(pin_out_h, prep_pon)
        net_h = _update(net_h, comb, p['net_up_w'], p['net_up_b'],
                        p['net_norm_g'], p['net_norm_b'])

    return _linear(net_h, p['out_w'], p['out_b'], act=False)


# speedup vs baseline: 5.9234x; 2.2073x over previous
import functools

import jax
import jax.numpy as jnp
from jax import lax
from jax.experimental import pallas as pl
from jax.experimental.pallas import tpu as pltpu
from jax.experimental.pallas import tpu_sc as plsc

HID = 64
EMB = 8
CT = 26
LAYERS = 4
ROW_BLK = 2000

NW = 32


WIDE = 128


def _linear_body(x_ref, w_ref, b_ref, *rest, act, k, o, x_off):
    if len(rest) == 2:
        s_ref, o_ref = rest
    else:
        s_ref, (o_ref,) = None, rest
    x = x_ref[...][:, x_off:x_off + k]
    y = jnp.dot(x, w_ref[...], preferred_element_type=jnp.float32) + b_ref[...]
    if act:
        y = jnp.maximum(y, 0.0)
    if s_ref is None:
        o_ref[:, :o] = y
    else:
        o_ref[:, :o] = y * s_ref[...]
        o_ref[:, HID:HID + o] = y


def _linear(x, w, b, act, scale=None, x_off=0):
    n, kin = x.shape
    k, o = w.shape
    grid = n // ROW_BLK
    specs = [
        pl.BlockSpec((ROW_BLK, kin), lambda i: (i, 0)),
        pl.BlockSpec((k, o), lambda i: (0, 0)),
        pl.BlockSpec((1, o), lambda i: (0, 0)),
    ]
    args = [x, w, b.reshape(1, o)]
    body = functools.partial(_linear_body, act=act, k=k, o=o, x_off=x_off)
    if scale is None:
        o_out = o
    else:
        specs.append(pl.BlockSpec((ROW_BLK, 1), lambda i: (i, 0)))
        args.append(scale)
        o_out = WIDE
    return pl.pallas_call(
        body,
        grid=(grid,),
        in_specs=specs,
        out_specs=pl.BlockSpec((ROW_BLK, o_out), lambda i: (i, 0)),
        out_shape=jax.ShapeDtypeStruct((n, o_out), jnp.float32),
    )(*args)


def _update_body(h_ref, c_ref, w_ref, b_ref, g_ref, beta_ref, s_ref, o_ref):
    h = h_ref[...][:, HID:2 * HID]
    x = h + jnp.dot(c_ref[...], w_ref[...], preferred_element_type=jnp.float32) + b_ref[...]
    m = jnp.mean(x, axis=-1, keepdims=True)
    v = jnp.mean((x - m) ** 2, axis=-1, keepdims=True)
    y = (x - m) * jax.lax.rsqrt(v + 1e-5) * g_ref[...] + beta_ref[...]
    y = jnp.maximum(y, 0.0)
    o_ref[:, :HID] = y * s_ref[...]
    o_ref[:, HID:] = y


def _update(h, comb, w, b, g, beta, scale):
    n = h.shape[0]
    grid = n // ROW_BLK
    return pl.pallas_call(
        _update_body,
        grid=(grid,),
        in_specs=[
            pl.BlockSpec((ROW_BLK, WIDE), lambda i: (i, 0)),
            pl.BlockSpec((ROW_BLK, 3 * HID), lambda i: (i, 0)),
            pl.BlockSpec((3 * HID, HID), lambda i: (0, 0)),
            pl.BlockSpec((1, HID), lambda i: (0, 0)),
            pl.BlockSpec((1, HID), lambda i: (0, 0)),
            pl.BlockSpec((1, HID), lambda i: (0, 0)),
            pl.BlockSpec((ROW_BLK, 1), lambda i: (i, 0)),
        ],
        out_specs=pl.BlockSpec((ROW_BLK, WIDE), lambda i: (i, 0)),
        out_shape=jax.ShapeDtypeStruct((n, WIDE), jnp.float32),
    )(h, comb, w, b.reshape(1, HID), g.reshape(1, HID), beta.reshape(1, HID),
      scale)


def _cell_mlp_body(h_ref, e_ref, w1a_ref, w1b_ref, b1_ref, w2_ref, b2_ref,
                   s_ref, o_ref):
    h = h_ref[...][:, HID:2 * HID]
    t = (jnp.dot(h, w1a_ref[...], preferred_element_type=jnp.float32)
         + jnp.dot(e_ref[...], w1b_ref[...], preferred_element_type=jnp.float32)
         + b1_ref[...])
    t = jnp.maximum(t, 0.0)
    y = jnp.dot(t, w2_ref[...], preferred_element_type=jnp.float32) + b2_ref[...]
    o_ref[:, :HID] = y * s_ref[...]
    o_ref[:, HID:] = y


def _cell_mlp(cell_h, cte, w1, b1, w2, b2, scale):
    n = cell_h.shape[0]
    grid = n // ROW_BLK
    w1a = w1[:HID]
    w1b = w1[HID:]
    return pl.pallas_call(
        _cell_mlp_body,
        grid=(grid,),
        in_specs=[
            pl.BlockSpec((ROW_BLK, WIDE), lambda i: (i, 0)),
            pl.BlockSpec((ROW_BLK, EMB), lambda i: (i, 0)),
            pl.BlockSpec((HID, HID), lambda i: (0, 0)),
            pl.BlockSpec((EMB, HID), lambda i: (0, 0)),
            pl.BlockSpec((1, HID), lambda i: (0, 0)),
            pl.BlockSpec((HID, HID), lambda i: (0, 0)),
            pl.BlockSpec((1, HID), lambda i: (0, 0)),
            pl.BlockSpec((ROW_BLK, 1), lambda i: (i, 0)),
        ],
        out_specs=pl.BlockSpec((ROW_BLK, WIDE), lambda i: (i, 0)),
        out_shape=jax.ShapeDtypeStruct((n, WIDE), jnp.float32),
    )(cell_h, cte, w1a, w1b, b1.reshape(1, HID), w2, b2.reshape(1, HID), scale)


def _make_sc_agg(n_src, d_tile, c_edge, r_out, num_cores=None, num_subcores=None,
                 interpret=False):
    mesh_kw = {}
    if num_cores is not None:
        mesh_kw = dict(num_cores=num_cores, num_subcores=num_subcores)
    mesh = plsc.VectorSubcoreMesh(core_axis_name="c", subcore_axis_name="s", **mesh_kw)
    n_pad = mesh.num_cores * mesh.num_subcores * d_tile
    rs_len = d_tile + 16

    @functools.partial(
        pl.kernel,
        out_type=jax.ShapeDtypeStruct((n_pad, 192), jnp.float32),
        mesh=mesh,
        interpret=interpret,
        compiler_params=None if interpret else pltpu.CompilerParams(
            needs_layout_passes=False, use_tc_tiling_on_sc=False),
        scratch_types=[
            pltpu.VMEM((2, c_edge), jnp.int32),
            pltpu.VMEM((2, c_edge, WIDE), jnp.float32),
            pltpu.VMEM((rs_len,), jnp.int32),
            pltpu.VMEM((r_out, 192), jnp.float32),
            pltpu.SemaphoreType.DMA,
            pltpu.SemaphoreType.DMA,
        ],
    )
    def agg(ed, rs, table, out, env_v, gat_v, rs_v, out_v, sem0, sem1):
        if interpret:
            wid = jnp.int32(0)
        else:
            wid = lax.axis_index("s") * mesh.num_cores + lax.axis_index("c")
        r0 = wid * d_tile
        pltpu.sync_copy(rs.at[pl.ds(r0, rs_len)], rs_v)
        e0 = rs_v[pl.ds(0, 16)][0]
        e1 = rs_v[pl.ds(d_tile, 16)][0]
        eb = (e0 // 128) * 128
        n_chunks = jnp.where(e1 > e0, (e1 - eb + c_edge - 1) // c_edge, 0)

        def idx_of(parity):
            return env_v[parity] if interpret else env_v.at[parity]

        def start(g, parity):
            pltpu.sync_copy(ed.at[pl.ds(eb + g * c_edge, c_edge)],
                            env_v.at[parity])
            sem = sem0 if parity == 0 else sem1
            pltpu.async_copy(table.at[idx_of(parity)], gat_v.at[parity], sem)

        def start_dyn(g):
            @pl.when(lax.rem(g, 2) == 0)
            def _():
                start(g, 0)

            @pl.when(lax.rem(g, 2) == 1)
            def _():
                start(g, 1)

        def wait_slot(parity):
            sem = sem0 if parity == 0 else sem1
            pltpu.make_async_copy(table.at[idx_of(parity)],
                                  gat_v.at[parity], sem).wait()

        @pl.when(n_chunks > 0)
        def _():
            start(0, 0)

        zeros = jnp.zeros((16,), jnp.float32)
        ninf = jnp.full((16,), -jnp.inf, jnp.float32)
        pinf = jnp.full((16,), jnp.inf, jnp.float32)

        def row_body(i, cur0):
            rv = rs_v[pl.ds(i, 16)]
            s_e = rv[0]
            t_e = rv[1]

            def edge_body(e, carry):
                (cur, a0, a1, a2, a3, x0, x1, x2, x3, n0, n1, n2, n3) = carry
                adv = e >= eb + (cur + 1) * c_edge

                @pl.when(adv)
                def _():
                    g_new = cur + 1

                    @pl.when(g_new + 1 < n_chunks)
                    def _():
                        start_dyn(g_new + 1)

                    @pl.when(lax.rem(g_new, 2) == 0)
                    def _():
                        wait_slot(0)

                    @pl.when(lax.rem(g_new, 2) == 1)
                    def _():
                        wait_slot(1)

                cur = jnp.where(adv, cur + 1, cur)
                el = e - eb - cur * c_edge
                slot = lax.rem(cur, 2)
                m0 = gat_v[slot, el, pl.ds(0, 16)]
                m1 = gat_v[slot, el, pl.ds(16, 16)]
                m2 = gat_v[slot, el, pl.ds(32, 16)]
                m3 = gat_v[slot, el, pl.ds(48, 16)]
                r0 = gat_v[slot, el, pl.ds(64, 16)]
                r1 = gat_v[slot, el, pl.ds(80, 16)]
                r2 = gat_v[slot, el, pl.ds(96, 16)]
                r3 = gat_v[slot, el, pl.ds(112, 16)]
                return (cur,
                        a0 + m0, a1 + m1, a2 + m2, a3 + m3,
                        jnp.maximum(x0, r0), jnp.maximum(x1, r1),
                        jnp.maximum(x2, r2), jnp.maximum(x3, r3),
                        jnp.minimum(n0, r0), jnp.minimum(n1, r1),
                        jnp.minimum(n2, r2), jnp.minimum(n3, r3))

            init = (cur0, zeros, zeros, zeros, zeros,
                    ninf, ninf, ninf, ninf, pinf, pinf, pinf, pinf)
            res = lax.fori_loop(s_e, t_e, edge_body, init)
            cur = res[0]
            has = t_e > s_e
            il = lax.rem(i, r_out)
            for j in range(4):
                out_v[il, pl.ds(16 * j, 16)] = res[1 + j]
                out_v[il, pl.ds(64 + 16 * j, 16)] = jnp.where(has, res[5 + j], 0.0)
                out_v[il, pl.ds(128 + 16 * j, 16)] = jnp.where(has, res[9 + j], 0.0)

            @pl.when(il == r_out - 1)
            def _():
                row = pl.multiple_of(r0 + i - (r_out - 1), r_out)
                pltpu.sync_copy(out_v, out.at[pl.ds(row, r_out)])

            return cur

        lax.fori_loop(0, d_tile, row_body, jnp.int32(-1))

    return agg


D_TILE = 1664
C_EDGE = 256
R_OUT = 128
N_PAD = NW * D_TILE


def _prep_relation(edge_index, n_src):
    src = edge_index[0]
    dst = edge_index[1]
    deg = jnp.maximum(jnp.bincount(src, length=n_src), 1)
    inv_deg = (1.0 / deg.astype(jnp.float32)).reshape(n_src, 1)
    order = jnp.argsort(dst)
    ed = jnp.pad(src[order], (0, C_EDGE))
    dst_s = dst[order]
    rs = jnp.searchsorted(dst_s, jnp.arange(N_PAD + 64, dtype=jnp.int32),
                          side='left').astype(jnp.int32)
    return ed, rs, inv_deg


_sc_agg = None


def _aggregate(src_x, prep):
    global _sc_agg
    if _sc_agg is None:
        _sc_agg = _make_sc_agg(src_x.shape[0], D_TILE, C_EDGE, R_OUT)
    ed, rs, _ = prep
    return _sc_agg(ed, rs, src_x)


def kernel(net_x, pin_in_x, pin_out_x, cell_x, edge_net_pin_in, edge_pin_in_cell,
           edge_cell_pin_out, edge_pin_out_net, params):
    p = params
    prep_npi = _prep_relation(edge_net_pin_in, net_x.shape[0])
    prep_pic = _prep_relation(edge_pin_in_cell, pin_in_x.shape[0])
    prep_cpo = _prep_relation(edge_cell_pin_out, cell_x.shape[0])
    prep_pon = _prep_relation(edge_pin_out_net, pin_out_x.shape[0])
    inv_npi, inv_pic, inv_cpo, inv_pon = (
        prep_npi[2], prep_pic[2], prep_cpo[2], prep_pon[2])
    ones_cell = jnp.ones((cell_x.shape[0], 1), jnp.float32)

    net_h = _linear(net_x, p['net_lin_w'], p['net_lin_b'], act=True, scale=inv_npi)
    pin_in_h = _linear(pin_in_x, p['pin_in_lin_w'], p['pin_in_lin_b'], act=True, scale=inv_pic)
    pin_out_h = _linear(pin_out_x, p['pin_out_lin_w'], p['pin_out_lin_b'], act=True, scale=inv_pon)
    cell_h = _linear(cell_x[:, CT:], p['cell_struct_w'], p['cell_struct_b'], act=True, scale=ones_cell)
    cte = _linear(cell_x[:, :CT], p['cte_w'], p['cte_b'], act=False)

    for _ in range(LAYERS):
        comb = _aggregate(net_h, prep_npi)
        pin_in_h = _update(pin_in_h, comb, p['pin_in_up_w'], p['pin_in_up_b'],
                           p['pin_in_norm_g'], p['pin_in_norm_b'], inv_pic)
        comb = _aggregate(pin_in_h, prep_pic)
        cell_h = _update(cell_h, comb, p['cell_up_w'], p['cell_up_b'],
                         p['cell_norm_g'], p['cell_norm_b'], ones_cell)
        cell_out = _cell_mlp(cell_h, cte, p['mlp1_w'], p['mlp1_b'],
                             p['mlp2_w'], p['mlp2_b'], inv_cpo)
        comb = _aggregate(cell_out, prep_cpo)
        pin_out_h = _update(pin_out_h, comb, p['pin_out_up_w'], p['pin_out_up_b'],
                            p['pin_out_norm_g'], p['pin_out_norm_b'], inv_pon)
        comb = _aggregate(pin_out_h, prep_pon)
        net_h = _update(net_h, comb, p['net_up_w'], p['net_up_b'],
                        p['net_norm_g'], p['net_norm_b'], inv_npi)

    return _linear(net_h, p['out_w'], p['out_b'], act=False, x_off=HID)


# speedup vs baseline: 7.7619x; 1.3104x over previous
import functools

import jax
import jax.numpy as jnp
from jax import lax
from jax.experimental import pallas as pl
from jax.experimental.pallas import tpu as pltpu
from jax.experimental.pallas import tpu_sc as plsc

HID = 64
EMB = 8
CT = 26
LAYERS = 4
ROW_BLK = 2000

NW = 32


WIDE = 128


def _linear_body(x_ref, w_ref, b_ref, *rest, act, k, o, x_off):
    if len(rest) == 2:
        s_ref, o_ref = rest
    else:
        s_ref, (o_ref,) = None, rest
    x = x_ref[...][:, x_off:x_off + k]
    y = jnp.dot(x, w_ref[...], preferred_element_type=jnp.float32) + b_ref[...]
    if act:
        y = jnp.maximum(y, 0.0)
    if s_ref is None:
        o_ref[:, :o] = y
    else:
        o_ref[:, :o] = y * s_ref[...]
        o_ref[:, HID:HID + o] = y


def _linear(x, w, b, act, scale=None, x_off=0):
    n, kin = x.shape
    k, o = w.shape
    grid = n // ROW_BLK
    specs = [
        pl.BlockSpec((ROW_BLK, kin), lambda i: (i, 0)),
        pl.BlockSpec((k, o), lambda i: (0, 0)),
        pl.BlockSpec((1, o), lambda i: (0, 0)),
    ]
    args = [x, w, b.reshape(1, o)]
    body = functools.partial(_linear_body, act=act, k=k, o=o, x_off=x_off)
    if scale is None:
        o_out = o
    else:
        specs.append(pl.BlockSpec((ROW_BLK, 1), lambda i: (i, 0)))
        args.append(scale)
        o_out = WIDE
    return pl.pallas_call(
        body,
        grid=(grid,),
        in_specs=specs,
        out_specs=pl.BlockSpec((ROW_BLK, o_out), lambda i: (i, 0)),
        out_shape=jax.ShapeDtypeStruct((n, o_out), jnp.float32),
    )(*args)


def _update_body(h_ref, c_ref, w_ref, b_ref, g_ref, beta_ref, s_ref, o_ref):
    h = h_ref[...][:, HID:2 * HID]
    x = h + jnp.dot(c_ref[...], w_ref[...], preferred_element_type=jnp.float32) + b_ref[...]
    m = jnp.mean(x, axis=-1, keepdims=True)
    v = jnp.mean((x - m) ** 2, axis=-1, keepdims=True)
    y = (x - m) * jax.lax.rsqrt(v + 1e-5) * g_ref[...] + beta_ref[...]
    y = jnp.maximum(y, 0.0)
    o_ref[:, :HID] = y * s_ref[...]
    o_ref[:, HID:] = y


def _update(h, comb, w, b, g, beta, scale):
    n = h.shape[0]
    grid = n // ROW_BLK
    return pl.pallas_call(
        _update_body,
        grid=(grid,),
        in_specs=[
            pl.BlockSpec((ROW_BLK, WIDE), lambda i: (i, 0)),
            pl.BlockSpec((ROW_BLK, 3 * HID), lambda i: (i, 0)),
            pl.BlockSpec((3 * HID, HID), lambda i: (0, 0)),
            pl.BlockSpec((1, HID), lambda i: (0, 0)),
            pl.BlockSpec((1, HID), lambda i: (0, 0)),
            pl.BlockSpec((1, HID), lambda i: (0, 0)),
            pl.BlockSpec((ROW_BLK, 1), lambda i: (i, 0)),
        ],
        out_specs=pl.BlockSpec((ROW_BLK, WIDE), lambda i: (i, 0)),
        out_shape=jax.ShapeDtypeStruct((n, WIDE), jnp.float32),
    )(h, comb, w, b.reshape(1, HID), g.reshape(1, HID), beta.reshape(1, HID),
      scale)


def _cell_mlp_body(h_ref, e_ref, w1a_ref, w1b_ref, b1_ref, w2_ref, b2_ref,
                   s_ref, o_ref):
    h = h_ref[...][:, HID:2 * HID]
    t = (jnp.dot(h, w1a_ref[...], preferred_element_type=jnp.float32)
         + jnp.dot(e_ref[...], w1b_ref[...], preferred_element_type=jnp.float32)
         + b1_ref[...])
    t = jnp.maximum(t, 0.0)
    y = jnp.dot(t, w2_ref[...], preferred_element_type=jnp.float32) + b2_ref[...]
    o_ref[:, :HID] = y * s_ref[...]
    o_ref[:, HID:] = y


def _cell_mlp(cell_h, cte, w1, b1, w2, b2, scale):
    n = cell_h.shape[0]
    grid = n // ROW_BLK
    w1a = w1[:HID]
    w1b = w1[HID:]
    return pl.pallas_call(
        _cell_mlp_body,
        grid=(grid,),
        in_specs=[
            pl.BlockSpec((ROW_BLK, WIDE), lambda i: (i, 0)),
            pl.BlockSpec((ROW_BLK, EMB), lambda i: (i, 0)),
            pl.BlockSpec((HID, HID), lambda i: (0, 0)),
            pl.BlockSpec((EMB, HID), lambda i: (0, 0)),
            pl.BlockSpec((1, HID), lambda i: (0, 0)),
            pl.BlockSpec((HID, HID), lambda i: (0, 0)),
            pl.BlockSpec((1, HID), lambda i: (0, 0)),
            pl.BlockSpec((ROW_BLK, 1), lambda i: (i, 0)),
        ],
        out_specs=pl.BlockSpec((ROW_BLK, WIDE), lambda i: (i, 0)),
        out_shape=jax.ShapeDtypeStruct((n, WIDE), jnp.float32),
    )(cell_h, cte, w1a, w1b, b1.reshape(1, HID), w2, b2.reshape(1, HID), scale)


def _make_sc_agg(n_src, d_tile, c_edge, r_out, num_cores=None, num_subcores=None,
                 interpret=False):
    mesh_kw = {}
    if num_cores is not None:
        mesh_kw = dict(num_cores=num_cores, num_subcores=num_subcores)
    mesh = plsc.VectorSubcoreMesh(core_axis_name="c", subcore_axis_name="s", **mesh_kw)
    n_pad = mesh.num_cores * mesh.num_subcores * d_tile
    rs_len = d_tile + 16

    @functools.partial(
        pl.kernel,
        out_type=jax.ShapeDtypeStruct((n_pad, 192), jnp.float32),
        mesh=mesh,
        interpret=interpret,
        compiler_params=None if interpret else pltpu.CompilerParams(
            needs_layout_passes=False, use_tc_tiling_on_sc=False),
        scratch_types=[
            pltpu.VMEM((2, c_edge), jnp.int32),
            pltpu.VMEM((2, c_edge, WIDE), jnp.float32),
            pltpu.VMEM((rs_len,), jnp.int32),
            pltpu.VMEM((r_out, 192), jnp.float32),
            pltpu.SemaphoreType.DMA,
            pltpu.SemaphoreType.DMA,
        ],
    )
    def agg(ed, rs, table, out, env_v, gat_v, rs_v, out_v, sem0, sem1):
        if interpret:
            wid = jnp.int32(0)
        else:
            wid = lax.axis_index("s") * mesh.num_cores + lax.axis_index("c")
        r0 = wid * d_tile
        pltpu.sync_copy(rs.at[pl.ds(r0, rs_len)], rs_v)
        e0 = rs_v[pl.ds(0, 16)][0]
        e1 = rs_v[pl.ds(d_tile, 16)][0]
        eb = (e0 // 128) * 128
        n_chunks = jnp.where(e1 > e0, (e1 - eb + c_edge - 1) // c_edge, 0)

        def idx_of(parity):
            return env_v[parity] if interpret else env_v.at[parity]

        def start(g, parity):
            pltpu.sync_copy(ed.at[pl.ds(eb + g * c_edge, c_edge)],
                            env_v.at[parity])
            sem = sem0 if parity == 0 else sem1
            pltpu.async_copy(table.at[idx_of(parity)], gat_v.at[parity], sem)

        def start_dyn(g):
            @pl.when(lax.rem(g, 2) == 0)
            def _():
                start(g, 0)

            @pl.when(lax.rem(g, 2) == 1)
            def _():
                start(g, 1)

        def wait_slot(parity):
            sem = sem0 if parity == 0 else sem1
            pltpu.make_async_copy(table.at[idx_of(parity)],
                                  gat_v.at[parity], sem).wait()

        @pl.when(n_chunks > 0)
        def _():
            start(0, 0)

        zeros = jnp.zeros((16,), jnp.float32)
        ninf = jnp.full((16,), -jnp.inf, jnp.float32)
        pinf = jnp.full((16,), jnp.inf, jnp.float32)

        def row_body(i, cur0):
            rv = rs_v[pl.ds(i, 16)]
            s_e = rv[0]
            t_e = rv[1]

            def edge_body(e, carry):
                (cur, a0, a1, a2, a3, x0, x1, x2, x3, n0, n1, n2, n3) = carry
                adv = e >= eb + (cur + 1) * c_edge

                @pl.when(adv)
                def _():
                    g_new = cur + 1

                    @pl.when(g_new + 1 < n_chunks)
                    def _():
                        start_dyn(g_new + 1)

                    @pl.when(lax.rem(g_new, 2) == 0)
                    def _():
                        wait_slot(0)

                    @pl.when(lax.rem(g_new, 2) == 1)
                    def _():
                        wait_slot(1)

                cur = jnp.where(adv, cur + 1, cur)
                el = e - eb - cur * c_edge
                slot = lax.rem(cur, 2)
                m0 = gat_v[slot, el, pl.ds(0, 16)]
                m1 = gat_v[slot, el, pl.ds(16, 16)]
                m2 = gat_v[slot, el, pl.ds(32, 16)]
                m3 = gat_v[slot, el, pl.ds(48, 16)]
                r0 = gat_v[slot, el, pl.ds(64, 16)]
                r1 = gat_v[slot, el, pl.ds(80, 16)]
                r2 = gat_v[slot, el, pl.ds(96, 16)]
                r3 = gat_v[slot, el, pl.ds(112, 16)]
                return (cur,
                        a0 + m0, a1 + m1, a2 + m2, a3 + m3,
                        jnp.maximum(x0, r0), jnp.maximum(x1, r1),
                        jnp.maximum(x2, r2), jnp.maximum(x3, r3),
                        jnp.minimum(n0, r0), jnp.minimum(n1, r1),
                        jnp.minimum(n2, r2), jnp.minimum(n3, r3))

            init = (cur0, zeros, zeros, zeros, zeros,
                    ninf, ninf, ninf, ninf, pinf, pinf, pinf, pinf)
            res = lax.fori_loop(s_e, t_e, edge_body, init)
            cur = res[0]
            has = t_e > s_e
            il = lax.rem(i, r_out)
            for j in range(4):
                out_v[il, pl.ds(16 * j, 16)] = res[1 + j]
                out_v[il, pl.ds(64 + 16 * j, 16)] = jnp.where(has, res[5 + j], 0.0)
                out_v[il, pl.ds(128 + 16 * j, 16)] = jnp.where(has, res[9 + j], 0.0)

            @pl.when(il == r_out - 1)
            def _():
                row = pl.multiple_of(r0 + i - (r_out - 1), r_out)
                pltpu.sync_copy(out_v, out.at[pl.ds(row, r_out)])

            return cur

        lax.fori_loop(0, d_tile, row_body, jnp.int32(-1))

    return agg


D_TILE = 1664
C_EDGE = 256
R_OUT = 128
N_PAD = NW * D_TILE


def _prep_relation(edge_index, n_src):
    src = edge_index[0]
    dst = edge_index[1]
    deg = jnp.maximum(jnp.bincount(src, length=n_src), 1)
    inv_deg = (1.0 / deg.astype(jnp.float32)).reshape(n_src, 1)
    _, src_s = lax.sort((dst, src), num_keys=1)
    ed = jnp.pad(src_s, (0, C_EDGE))
    counts = jnp.bincount(dst, length=N_PAD + 64)
    rs = (jnp.cumsum(counts) - counts).astype(jnp.int32)
    return ed, rs, inv_deg


_sc_agg = None


def _aggregate(src_x, prep):
    global _sc_agg
    if _sc_agg is None:
        _sc_agg = _make_sc_agg(src_x.shape[0], D_TILE, C_EDGE, R_OUT)
    ed, rs, _ = prep
    return _sc_agg(ed, rs, src_x)


def kernel(net_x, pin_in_x, pin_out_x, cell_x, edge_net_pin_in, edge_pin_in_cell,
           edge_cell_pin_out, edge_pin_out_net, params):
    p = params
    prep_npi = _prep_relation(edge_net_pin_in, net_x.shape[0])
    prep_pic = _prep_relation(edge_pin_in_cell, pin_in_x.shape[0])
    prep_cpo = _prep_relation(edge_cell_pin_out, cell_x.shape[0])
    prep_pon = _prep_relation(edge_pin_out_net, pin_out_x.shape[0])
    inv_npi, inv_pic, inv_cpo, inv_pon = (
        prep_npi[2], prep_pic[2], prep_cpo[2], prep_pon[2])
    ones_cell = jnp.ones((cell_x.shape[0], 1), jnp.float32)

    net_h = _linear(net_x, p['net_lin_w'], p['net_lin_b'], act=True, scale=inv_npi)
    pin_in_h = _linear(pin_in_x, p['pin_in_lin_w'], p['pin_in_lin_b'], act=True, scale=inv_pic)
    pin_out_h = _linear(pin_out_x, p['pin_out_lin_w'], p['pin_out_lin_b'], act=True, scale=inv_pon)
    cell_h = _linear(cell_x[:, CT:], p['cell_struct_w'], p['cell_struct_b'], act=True, scale=ones_cell)
    cte = _linear(cell_x[:, :CT], p['cte_w'], p['cte_b'], act=False)

    for _ in range(LAYERS):
        comb = _aggregate(net_h, prep_npi)
        pin_in_h = _update(pin_in_h, comb, p['pin_in_up_w'], p['pin_in_up_b'],
                           p['pin_in_norm_g'], p['pin_in_norm_b'], inv_pic)
        comb = _aggregate(pin_in_h, prep_pic)
        cell_h = _update(cell_h, comb, p['cell_up_w'], p['cell_up_b'],
                         p['cell_norm_g'], p['cell_norm_b'], ones_cell)
        cell_out = _cell_mlp(cell_h, cte, p['mlp1_w'], p['mlp1_b'],
                             p['mlp2_w'], p['mlp2_b'], inv_cpo)
        comb = _aggregate(cell_out, prep_cpo)
        pin_out_h = _update(pin_out_h, comb, p['pin_out_up_w'], p['pin_out_up_b'],
                            p['pin_out_norm_g'], p['pin_out_norm_b'], inv_pon)
        comb = _aggregate(pin_out_h, prep_pon)
        net_h = _update(net_h, comb, p['net_up_w'], p['net_up_b'],
                        p['net_norm_g'], p['net_norm_b'], inv_npi)

    return _linear(net_h, p['out_w'], p['out_b'], act=False, x_off=HID)


# speedup vs baseline: 7.8668x; 1.0135x over previous
import functools

import jax
import jax.numpy as jnp
from jax import lax
from jax.experimental import pallas as pl
from jax.experimental.pallas import tpu as pltpu
from jax.experimental.pallas import tpu_sc as plsc

HID = 64
EMB = 8
CT = 26
LAYERS = 4
ROW_BLK = 2000

NW = 32


WIDE = 128


def _linear_body(x_ref, w_ref, b_ref, *rest, act, k, o, x_off):
    if len(rest) == 2:
        s_ref, o_ref = rest
    else:
        s_ref, (o_ref,) = None, rest
    x = x_ref[...][:, x_off:x_off + k]
    y = jnp.dot(x, w_ref[...], preferred_element_type=jnp.float32) + b_ref[...]
    if act:
        y = jnp.maximum(y, 0.0)
    if s_ref is None:
        o_ref[:, :o] = y
    else:
        o_ref[:, :o] = y * s_ref[...]
        o_ref[:, HID:HID + o] = y


def _linear(x, w, b, act, scale=None, x_off=0):
    n, kin = x.shape
    k, o = w.shape
    grid = n // ROW_BLK
    specs = [
        pl.BlockSpec((ROW_BLK, kin), lambda i: (i, 0)),
        pl.BlockSpec((k, o), lambda i: (0, 0)),
        pl.BlockSpec((1, o), lambda i: (0, 0)),
    ]
    args = [x, w, b.reshape(1, o)]
    body = functools.partial(_linear_body, act=act, k=k, o=o, x_off=x_off)
    if scale is None:
        o_out = o
    else:
        specs.append(pl.BlockSpec((ROW_BLK, 1), lambda i: (i, 0)))
        args.append(scale)
        o_out = WIDE
    return pl.pallas_call(
        body,
        grid=(grid,),
        in_specs=specs,
        out_specs=pl.BlockSpec((ROW_BLK, o_out), lambda i: (i, 0)),
        out_shape=jax.ShapeDtypeStruct((n, o_out), jnp.float32),
    )(*args)


def _update_body(h_ref, c_ref, w_ref, b_ref, g_ref, beta_ref, s_ref, o_ref):
    h = h_ref[...][:, HID:2 * HID]
    x = h + jnp.dot(c_ref[...], w_ref[...], preferred_element_type=jnp.float32) + b_ref[...]
    m = jnp.mean(x, axis=-1, keepdims=True)
    v = jnp.mean((x - m) ** 2, axis=-1, keepdims=True)
    y = (x - m) * jax.lax.rsqrt(v + 1e-5) * g_ref[...] + beta_ref[...]
    y = jnp.maximum(y, 0.0)
    o_ref[:, :HID] = y * s_ref[...]
    o_ref[:, HID:] = y


def _update(h, comb, w, b, g, beta, scale):
    n = h.shape[0]
    grid = n // ROW_BLK
    return pl.pallas_call(
        _update_body,
        grid=(grid,),
        in_specs=[
            pl.BlockSpec((ROW_BLK, WIDE), lambda i: (i, 0)),
            pl.BlockSpec((ROW_BLK, 3 * HID), lambda i: (i, 0)),
            pl.BlockSpec((3 * HID, HID), lambda i: (0, 0)),
            pl.BlockSpec((1, HID), lambda i: (0, 0)),
            pl.BlockSpec((1, HID), lambda i: (0, 0)),
            pl.BlockSpec((1, HID), lambda i: (0, 0)),
            pl.BlockSpec((ROW_BLK, 1), lambda i: (i, 0)),
        ],
        out_specs=pl.BlockSpec((ROW_BLK, WIDE), lambda i: (i, 0)),
        out_shape=jax.ShapeDtypeStruct((n, WIDE), jnp.float32),
    )(h, comb, w, b.reshape(1, HID), g.reshape(1, HID), beta.reshape(1, HID),
      scale)


def _cell_mlp_body(h_ref, e_ref, w1a_ref, w1b_ref, b1_ref, w2_ref, b2_ref,
                   s_ref, o_ref):
    h = h_ref[...][:, HID:2 * HID]
    t = (jnp.dot(h, w1a_ref[...], preferred_element_type=jnp.float32)
         + jnp.dot(e_ref[...], w1b_ref[...], preferred_element_type=jnp.float32)
         + b1_ref[...])
    t = jnp.maximum(t, 0.0)
    y = jnp.dot(t, w2_ref[...], preferred_element_type=jnp.float32) + b2_ref[...]
    o_ref[:, :HID] = y * s_ref[...]
    o_ref[:, HID:] = y


def _cell_mlp(cell_h, cte, w1, b1, w2, b2, scale):
    n = cell_h.shape[0]
    grid = n // ROW_BLK
    w1a = w1[:HID]
    w1b = w1[HID:]
    return pl.pallas_call(
        _cell_mlp_body,
        grid=(grid,),
        in_specs=[
            pl.BlockSpec((ROW_BLK, WIDE), lambda i: (i, 0)),
            pl.BlockSpec((ROW_BLK, EMB), lambda i: (i, 0)),
            pl.BlockSpec((HID, HID), lambda i: (0, 0)),
            pl.BlockSpec((EMB, HID), lambda i: (0, 0)),
            pl.BlockSpec((1, HID), lambda i: (0, 0)),
            pl.BlockSpec((HID, HID), lambda i: (0, 0)),
            pl.BlockSpec((1, HID), lambda i: (0, 0)),
            pl.BlockSpec((ROW_BLK, 1), lambda i: (i, 0)),
        ],
        out_specs=pl.BlockSpec((ROW_BLK, WIDE), lambda i: (i, 0)),
        out_shape=jax.ShapeDtypeStruct((n, WIDE), jnp.float32),
    )(cell_h, cte, w1a, w1b, b1.reshape(1, HID), w2, b2.reshape(1, HID), scale)


def _make_sc_agg(n_src, d_tile, c_edge, r_out, num_cores=None, num_subcores=None,
                 interpret=False):
    mesh_kw = {}
    if num_cores is not None:
        mesh_kw = dict(num_cores=num_cores, num_subcores=num_subcores)
    mesh = plsc.VectorSubcoreMesh(core_axis_name="c", subcore_axis_name="s", **mesh_kw)
    n_pad = mesh.num_cores * mesh.num_subcores * d_tile
    rs_len = d_tile + 16

    @functools.partial(
        pl.kernel,
        out_type=jax.ShapeDtypeStruct((n_pad, 192), jnp.float32),
        mesh=mesh,
        interpret=interpret,
        compiler_params=None if interpret else pltpu.CompilerParams(
            needs_layout_passes=False, use_tc_tiling_on_sc=False),
        scratch_types=[
            pltpu.VMEM((2, c_edge), jnp.int32),
            pltpu.VMEM((2, c_edge, WIDE), jnp.float32),
            pltpu.VMEM((rs_len,), jnp.int32),
            pltpu.VMEM((r_out, 192), jnp.float32),
            pltpu.SemaphoreType.DMA,
            pltpu.SemaphoreType.DMA,
        ],
    )
    def agg(ed, rs, table, out, env_v, gat_v, rs_v, out_v, sem0, sem1):
        if interpret:
            wid = jnp.int32(0)
        else:
            wid = lax.axis_index("s") * mesh.num_cores + lax.axis_index("c")
        r0 = wid * d_tile
        pltpu.sync_copy(rs.at[pl.ds(r0, rs_len)], rs_v)
        e0 = rs_v[pl.ds(0, 16)][0]
        e1 = rs_v[pl.ds(d_tile, 16)][0]
        eb = (e0 // 128) * 128
        n_chunks = jnp.where(e1 > e0, (e1 - eb + c_edge - 1) // c_edge, 0)

        def idx_of(parity):
            return env_v[parity] if interpret else env_v.at[parity]

        def start(g, parity):
            pltpu.sync_copy(ed.at[pl.ds(eb + g * c_edge, c_edge)],
                            env_v.at[parity])
            sem = sem0 if parity == 0 else sem1
            pltpu.async_copy(table.at[idx_of(parity)], gat_v.at[parity], sem)

        def start_dyn(g):
            @pl.when(lax.rem(g, 2) == 0)
            def _():
                start(g, 0)

            @pl.when(lax.rem(g, 2) == 1)
            def _():
                start(g, 1)

        def wait_slot(parity):
            sem = sem0 if parity == 0 else sem1
            pltpu.make_async_copy(table.at[idx_of(parity)],
                                  gat_v.at[parity], sem).wait()

        @pl.when(n_chunks > 0)
        def _():
            start(0, 0)

        zeros = jnp.zeros((16,), jnp.float32)
        ninf = jnp.full((16,), -jnp.inf, jnp.float32)
        pinf = jnp.full((16,), jnp.inf, jnp.float32)

        def row_body(i, chunk0):
            cur0, hi0 = chunk0
            rv = rs_v[pl.ds(i, 16)]
            s_e = rv[0]
            t_e = rv[1]

            def edge_body(e, carry):
                (cur, hi, a0, a1, a2, a3, x0, x1, x2, x3, n0, n1, n2, n3) = carry
                adv = e >= hi

                @pl.when(adv)
                def _():
                    g_new = cur + 1

                    @pl.when(g_new + 1 < n_chunks)
                    def _():
                        start_dyn(g_new + 1)

                    @pl.when(lax.rem(g_new, 2) == 0)
                    def _():
                        wait_slot(0)

                    @pl.when(lax.rem(g_new, 2) == 1)
                    def _():
                        wait_slot(1)

                cur = jnp.where(adv, cur + 1, cur)
                hi = jnp.where(adv, hi + c_edge, hi)
                el = e - hi + c_edge
                slot = lax.rem(cur, 2)
                m0 = gat_v[slot, el, pl.ds(0, 16)]
                m1 = gat_v[slot, el, pl.ds(16, 16)]
                m2 = gat_v[slot, el, pl.ds(32, 16)]
                m3 = gat_v[slot, el, pl.ds(48, 16)]
                r0 = gat_v[slot, el, pl.ds(64, 16)]
                r1 = gat_v[slot, el, pl.ds(80, 16)]
                r2 = gat_v[slot, el, pl.ds(96, 16)]
                r3 = gat_v[slot, el, pl.ds(112, 16)]
                return (cur, hi,
                        a0 + m0, a1 + m1, a2 + m2, a3 + m3,
                        jnp.maximum(x0, r0), jnp.maximum(x1, r1),
                        jnp.maximum(x2, r2), jnp.maximum(x3, r3),
                        jnp.minimum(n0, r0), jnp.minimum(n1, r1),
                        jnp.minimum(n2, r2), jnp.minimum(n3, r3))

            init = (cur0, hi0, zeros, zeros, zeros, zeros,
                    ninf, ninf, ninf, ninf, pinf, pinf, pinf, pinf)
            res = lax.fori_loop(s_e, t_e, edge_body, init)
            has = t_e > s_e
            il = lax.rem(i, r_out)
            for j in range(4):
                out_v[il, pl.ds(16 * j, 16)] = res[2 + j]
                out_v[il, pl.ds(64 + 16 * j, 16)] = jnp.where(has, res[6 + j], 0.0)
                out_v[il, pl.ds(128 + 16 * j, 16)] = jnp.where(has, res[10 + j], 0.0)

            @pl.when(il == r_out - 1)
            def _():
                row = pl.multiple_of(r0 + i - (r_out - 1), r_out)
                pltpu.sync_copy(out_v, out.at[pl.ds(row, r_out)])

            return (res[0], res[1])

        lax.fori_loop(0, d_tile, row_body, (jnp.int32(-1), eb))

    return agg


D_TILE = 1664
C_EDGE = 256
R_OUT = 128
N_PAD = NW * D_TILE


def _prep_relation(edge_index, n_src):
    src = edge_index[0]
    dst = edge_index[1]
    deg = jnp.maximum(jnp.bincount(src, length=n_src), 1)
    inv_deg = (1.0 / deg.astype(jnp.float32)).reshape(n_src, 1)
    _, src_s = lax.sort((dst, src), num_keys=1)
    ed = jnp.pad(src_s, (0, C_EDGE))
    counts = jnp.bincount(dst, length=N_PAD + 64)
    rs = (jnp.cumsum(counts) - counts).astype(jnp.int32)
    return ed, rs, inv_deg


_sc_agg = None


def _aggregate(src_x, prep):
    global _sc_agg
    if _sc_agg is None:
        _sc_agg = _make_sc_agg(src_x.shape[0], D_TILE, C_EDGE, R_OUT)
    ed, rs, _ = prep
    return _sc_agg(ed, rs, src_x)


def kernel(net_x, pin_in_x, pin_out_x, cell_x, edge_net_pin_in, edge_pin_in_cell,
           edge_cell_pin_out, edge_pin_out_net, params):
    p = params
    prep_npi = _prep_relation(edge_net_pin_in, net_x.shape[0])
    prep_pic = _prep_relation(edge_pin_in_cell, pin_in_x.shape[0])
    prep_cpo = _prep_relation(edge_cell_pin_out, cell_x.shape[0])
    prep_pon = _prep_relation(edge_pin_out_net, pin_out_x.shape[0])
    inv_npi, inv_pic, inv_cpo, inv_pon = (
        prep_npi[2], prep_pic[2], prep_cpo[2], prep_pon[2])
    ones_cell = jnp.ones((cell_x.shape[0], 1), jnp.float32)

    net_h = _linear(net_x, p['net_lin_w'], p['net_lin_b'], act=True, scale=inv_npi)
    pin_in_h = _linear(pin_in_x, p['pin_in_lin_w'], p['pin_in_lin_b'], act=True, scale=inv_pic)
    pin_out_h = _linear(pin_out_x, p['pin_out_lin_w'], p['pin_out_lin_b'], act=True, scale=inv_pon)
    cell_h = _linear(cell_x[:, CT:], p['cell_struct_w'], p['cell_struct_b'], act=True, scale=ones_cell)
    cte = _linear(cell_x[:, :CT], p['cte_w'], p['cte_b'], act=False)

    for _ in range(LAYERS):
        comb = _aggregate(net_h, prep_npi)
        pin_in_h = _update(pin_in_h, comb, p['pin_in_up_w'], p['pin_in_up_b'],
                           p['pin_in_norm_g'], p['pin_in_norm_b'], inv_pic)
        comb = _aggregate(pin_in_h, prep_pic)
        cell_h = _update(cell_h, comb, p['cell_up_w'], p['cell_up_b'],
                         p['cell_norm_g'], p['cell_norm_b'], ones_cell)
        cell_out = _cell_mlp(cell_h, cte, p['mlp1_w'], p['mlp1_b'],
                             p['mlp2_w'], p['mlp2_b'], inv_cpo)
        comb = _aggregate(cell_out, prep_cpo)
        pin_out_h = _update(pin_out_h, comb, p['pin_out_up_w'], p['pin_out_up_b'],
                            p['pin_out_norm_g'], p['pin_out_norm_b'], inv_pon)
        comb = _aggregate(pin_out_h, prep_pon)
        net_h = _update(net_h, comb, p['net_up_w'], p['net_up_b'],
                        p['net_norm_g'], p['net_norm_b'], inv_npi)

    return _linear(net_h, p['out_w'], p['out_b'], act=False, x_off=HID)
